# baseline (speedup 1.0000x reference)
import jax
import jax.numpy as jnp
from jax.experimental import pallas as pl

ALPHA = 0.2
N = 512
F = 64
LOG2E = 1.4426950408889634


def _attention(h, adj, a_lo, a_hi):
    a_both = jnp.concatenate([a_lo, a_hi], axis=1)
    ones_col = jnp.ones((N, 1), jnp.float32)
    ff = jnp.dot(h, a_both, preferred_element_type=jnp.float32)
    lhs = jnp.concatenate([ff[:, 0:1], ones_col], axis=1)
    rhs = jnp.concatenate([ones_col, ff[:, 1:2]], axis=1)
    v = jax.lax.dot_general(lhs, rhs, (((1,), (1,)), ((), ())),
                            preferred_element_type=jnp.float32)
    p = adj * jnp.exp2(jnp.maximum(v, ALPHA * v))
    he = jnp.concatenate([h, ones_col], axis=1)
    num = jnp.dot(p, he, preferred_element_type=jnp.float32)
    return num[:, :-1] * (1.0 / num[:, -1:])


def _gatcell_kernel(x_ref, adj_ref, w1_ref, a1_ref, w2_ref, a2_ref, out_ref):
    adj = adj_ref[...]
    x = x_ref[0]

    w1eff = w1_ref[:F, :] + w1_ref[F:, :]
    h1 = jnp.dot(x, w1eff, preferred_element_type=jnp.float32)
    gv = _attention(h1, adj, LOG2E * a1_ref[:2 * F, :],
                    LOG2E * a1_ref[2 * F:, :])

    r = jax.nn.sigmoid(gv[:, :F])
    z = jax.nn.sigmoid(gv[:, F:])
    h2 = (jnp.dot(x, w2_ref[:F, :], preferred_element_type=jnp.float32)
          + jnp.dot(r * x, w2_ref[F:, :], preferred_element_type=jnp.float32))
    hp = _attention(h2, adj, LOG2E * a2_ref[:F, :], LOG2E * a2_ref[F:, :])

    t = jnp.tanh(hp)
    out_ref[0] = t + z * (x - t)


def kernel(X, adj, W1, a1, W2, a2):
    b, n, f = X.shape
    return pl.pallas_call(
        _gatcell_kernel,
        grid=(b,),
        in_specs=[
            pl.BlockSpec((1, n, f), lambda i: (i, 0, 0)),
            pl.BlockSpec(adj.shape, lambda i: (0, 0)),
            pl.BlockSpec(W1.shape, lambda i: (0, 0)),
            pl.BlockSpec(a1.shape, lambda i: (0, 0)),
            pl.BlockSpec(W2.shape, lambda i: (0, 0)),
            pl.BlockSpec(a2.shape, lambda i: (0, 0)),
        ],
        out_specs=pl.BlockSpec((1, n, f), lambda i: (i, 0, 0)),
        out_shape=jax.ShapeDtypeStruct(X.shape, X.dtype),
    )(X, adj, W1, a1, W2, a2)


# speedup vs baseline: 1.1172x; 1.1172x over previous
import jax
import jax.numpy as jnp
from jax.experimental import pallas as pl

ALPHA = 0.2
N = 512
F = 64
B = 2
LOG2E = 1.4426950408889634


def _attention(hs, adj, a_lo, a_hi):
    a_both = jnp.concatenate([a_lo, a_hi], axis=1)
    ones_col = jnp.ones((N, 1), jnp.float32)
    outs = []
    for h in hs:
        ff = jnp.dot(h, a_both, preferred_element_type=jnp.float32)
        lhs = jnp.concatenate([ff[:, 0:1], ones_col], axis=1)
        rhs = jnp.concatenate([ones_col, ff[:, 1:2]], axis=1)
        v = jax.lax.dot_general(lhs, rhs, (((1,), (1,)), ((), ())),
                                preferred_element_type=jnp.float32)
        p = adj * jnp.exp2(jnp.maximum(v, ALPHA * v))
        he = jnp.concatenate([h, ones_col], axis=1)
        num = jnp.dot(p, he, preferred_element_type=jnp.float32)
        outs.append(num[:, :-1] * (1.0 / num[:, -1:]))
    return outs


def _gatcell_kernel(x_ref, adj_ref, w1_ref, a1_ref, w2_ref, a2_ref, out_ref):
    adj = adj_ref[...]
    xs = [x_ref[b] for b in range(B)]

    w1eff = w1_ref[:F, :] + w1_ref[F:, :]
    h1s = [jnp.dot(x, w1eff, preferred_element_type=jnp.float32) for x in xs]
    gvs = _attention(h1s, adj, LOG2E * a1_ref[:2 * F, :],
                     LOG2E * a1_ref[2 * F:, :])

    rs_zs = [(jax.nn.sigmoid(gv[:, :F]), jax.nn.sigmoid(gv[:, F:]))
             for gv in gvs]
    h2s = [jnp.dot(x, w2_ref[:F, :], preferred_element_type=jnp.float32)
           + jnp.dot(r * x, w2_ref[F:, :], preferred_element_type=jnp.float32)
           for x, (r, _) in zip(xs, rs_zs)]
    hps = _attention(h2s, adj, LOG2E * a2_ref[:F, :], LOG2E * a2_ref[F:, :])

    for b, (x, (_, z), hp) in enumerate(zip(xs, rs_zs, hps)):
        t = jnp.tanh(hp)
        out_ref[b] = t + z * (x - t)


def kernel(X, adj, W1, a1, W2, a2):
    return pl.pallas_call(
        _gatcell_kernel,
        out_shape=jax.ShapeDtypeStruct(X.shape, X.dtype),
    )(X, adj, W1, a1, W2, a2)
